# baseline (speedup 1.0000x reference)
import jax
import jax.numpy as jnp
from jax.experimental import pallas as pl
from jax.experimental.pallas import tpu as pltpu

B = 16
T = 4
P = 1025
H = 1280
R = 9


def _body(ids_ref, gate_ref, h_ref, emb_ref, o_ref):
    g = jnp.tanh(gate_ref[0])
    o_ref[...] = h_ref[...] + emb_ref[0] * g


def kernel(hidden_state, aspect_ratio_ids, embedding_weight, gate):
    ids = aspect_ratio_ids.astype(jnp.int32)
    h = hidden_state.reshape(B * T, P, H)
    emb = embedding_weight.reshape(R, T, 1, H)

    grid_spec = pltpu.PrefetchScalarGridSpec(
        num_scalar_prefetch=2,
        grid=(B * T,),
        in_specs=[
            pl.BlockSpec((1, P, H), lambda i, ids, gate: (i, 0, 0)),
            pl.BlockSpec((1, 1, 1, H), lambda i, ids, gate: (ids[i // T], i % T, 0, 0)),
        ],
        out_specs=pl.BlockSpec((1, P, H), lambda i, ids, gate: (i, 0, 0)),
    )

    out = pl.pallas_call(
        _body,
        grid_spec=grid_spec,
        out_shape=jax.ShapeDtypeStruct((B * T, P, H), jnp.float32),
    )(ids, gate, h, emb)
    return out.reshape(B, T, P, H)


# speedup vs baseline: 3.3198x; 3.3198x over previous
import jax
import jax.numpy as jnp
from jax.experimental import pallas as pl
from jax.experimental.pallas import tpu as pltpu

B = 16
T = 4
P = 1025
H = 1280
R = 9


def _body(ids_ref, gate_ref, h_ref, emb_ref, o_ref):
    b = pl.program_id(0)
    t = pl.program_id(1)
    row = ids_ref[b]
    g = jnp.tanh(gate_ref[0])
    e = emb_ref[row, pl.ds(t * H, H)]
    o_ref[...] = h_ref[...] + e * g


def kernel(hidden_state, aspect_ratio_ids, embedding_weight, gate):
    ids = aspect_ratio_ids.astype(jnp.int32)

    grid_spec = pltpu.PrefetchScalarGridSpec(
        num_scalar_prefetch=2,
        grid=(B, T),
        in_specs=[
            pl.BlockSpec((1, 1, P, H), lambda b, t, ids, gate: (b, t, 0, 0)),
            pl.BlockSpec((R, T * H), lambda b, t, ids, gate: (0, 0)),
        ],
        out_specs=pl.BlockSpec((1, 1, P, H), lambda b, t, ids, gate: (b, t, 0, 0)),
    )

    return pl.pallas_call(
        _body,
        grid_spec=grid_spec,
        out_shape=jax.ShapeDtypeStruct((B, T, P, H), jnp.float32),
    )(ids, gate, hidden_state, embedding_weight)
